# baseline (device time: 25761 ns/iter reference)
import jax
import jax.numpy as jnp
from jax import lax
from jax.experimental import pallas as pl
from jax.experimental.pallas import tpu as pltpu

N_DEV = 4


def kernel(x, w_mat):
    m_per, k = x.shape
    _, n_per = w_mat.shape
    half = m_per // 2

    def body(x_ref, w_ref, out_ref, buf_l, buf_r, buf_d, sems):
        my_pos = lax.axis_index("i")
        left = (my_pos - 1) % N_DEV
        right = (my_pos + 1) % N_DEV

        barrier_sem = pltpu.get_barrier_semaphore()
        for nbr in (left, right):
            pl.semaphore_signal(
                barrier_sem, inc=1,
                device_id=(nbr,), device_id_type=pl.DeviceIdType.MESH,
            )
        pl.semaphore_wait(barrier_sem, 2)

        def remote(src, dst, s, dev):
            return pltpu.make_async_remote_copy(
                src_ref=src, dst_ref=dst,
                send_sem=sems.at[s], recv_sem=sems.at[s + 1],
                device_id=(dev,), device_id_type=pl.DeviceIdType.MESH,
            )

        top = pl.ds(0, half)
        bot = pl.ds(half, half)

        sr_top = remote(x_ref.at[top, :], buf_l.at[top, :], 0, right)
        sr_bot = remote(x_ref.at[bot, :], buf_l.at[bot, :], 2, right)
        sl_bot = remote(x_ref.at[bot, :], buf_r.at[bot, :], 4, left)
        sl_top = remote(x_ref.at[top, :], buf_r.at[top, :], 6, left)
        sr_top.start()
        sl_bot.start()
        sr_bot.start()
        sl_top.start()

        out_ref[pl.ds(my_pos * m_per, m_per), :] = jnp.dot(
            x_ref[...], w_ref[...], preferred_element_type=jnp.float32
        )

        sr_top.wait_recv()
        fwd_r = remote(buf_l.at[top, :], buf_d.at[top, :], 8, right)
        fwd_r.start()

        sl_bot.wait_recv()
        fwd_l = remote(buf_r.at[bot, :], buf_d.at[bot, :], 10, left)
        fwd_l.start()

        sr_bot.wait_recv()
        origin_l = (my_pos - 1) % N_DEV
        out_ref[pl.ds(origin_l * m_per, m_per), :] = jnp.dot(
            buf_l[...], w_ref[...], preferred_element_type=jnp.float32
        )
        sl_top.wait_recv()
        origin_r = (my_pos + 1) % N_DEV
        out_ref[pl.ds(origin_r * m_per, m_per), :] = jnp.dot(
            buf_r[...], w_ref[...], preferred_element_type=jnp.float32
        )

        fwd_r.wait_recv()
        fwd_l.wait_recv()
        origin_d = (my_pos + 2) % N_DEV
        out_ref[pl.ds(origin_d * m_per, m_per), :] = jnp.dot(
            buf_d[...], w_ref[...], preferred_element_type=jnp.float32
        )

        for rdma in (sr_top, sr_bot, sl_bot, sl_top, fwd_r, fwd_l):
            rdma.wait_send()

    return pl.pallas_call(
        body,
        out_shape=jax.ShapeDtypeStruct((N_DEV * m_per, n_per), jnp.float32),
        in_specs=[
            pl.BlockSpec(memory_space=pltpu.VMEM),
            pl.BlockSpec(memory_space=pltpu.VMEM),
        ],
        out_specs=pl.BlockSpec(memory_space=pltpu.VMEM),
        scratch_shapes=[
            pltpu.VMEM((m_per, k), x.dtype),
            pltpu.VMEM((m_per, k), x.dtype),
            pltpu.VMEM((m_per, k), x.dtype),
            pltpu.SemaphoreType.DMA((12,)),
        ],
        compiler_params=pltpu.CompilerParams(collective_id=0),
    )(x, w_mat)
